# baseline (speedup 1.0000x reference)
import functools

import jax
import jax.numpy as jnp
from jax import lax
from jax.experimental import pallas as pl
from jax.experimental.pallas import tpu as pltpu
from jax.experimental.pallas import tpu_sc as plsc


def _build_gather(V, D, N, NC, NS):
  NW = NC * NS
  NR = N // NW
  CH = 400
  NCH = NR // CH
  NBUF = 4
  K = 80
  NG = CH // K
  DP = 2 * D
  assert N % NW == 0 and NR % CH == 0 and NCH % NBUF == 0 and CH % K == 0
  assert K % 8 == 0

  mesh = plsc.VectorSubcoreMesh(core_axis_name="c", subcore_axis_name="s")

  scratch = (
      [pltpu.VMEM((CH, D), jnp.float32) for _ in range(NBUF)]
      + [pltpu.VMEM((CH,), jnp.int32) for _ in range(NBUF)]
      + [pltpu.SemaphoreType.DMA for _ in range(2 * NBUF)]
  )

  @functools.partial(
      pl.kernel,
      mesh=mesh,
      out_type=jax.ShapeDtypeStruct((N, DP), jnp.float32),
      scratch_types=scratch,
      compiler_params=pltpu.CompilerParams(use_tc_tiling_on_sc=False),
  )
  def gather(table, idx_hbm, out_hbm, *scr):
    rows = scr[0:NBUF]
    idxb = scr[NBUF:2 * NBUF]
    gsem = scr[2 * NBUF: 3 * NBUF]
    osem = scr[3 * NBUF: 4 * NBUF]

    wid = lax.axis_index("s") * NC + lax.axis_index("c")
    base = wid * NR

    def fire_gathers(g, b):
      r0 = base + g * CH
      pltpu.sync_copy(idx_hbm.at[pl.ds(r0, CH)], idxb[b])
      for kk in range(NG):
        pltpu.async_copy(
            table.at[idxb[b].at[pl.ds(kk * K, K)]],
            rows[b].at[pl.ds(kk * K, K)],
            gsem[b],
        )

    def wait_gathers(b):
      for kk in range(NG):
        pltpu.make_async_copy(
            table.at[idxb[b].at[pl.ds(kk * K, K)]],
            rows[b].at[pl.ds(kk * K, K)],
            gsem[b],
        ).wait()

    def out_slice(g):
      return out_hbm.at[pl.ds(base + g * CH, CH), pl.ds(0, D)]

    def wait_outwrite(g, b):
      pltpu.make_async_copy(rows[b], out_slice(g), osem[b]).wait()

    for b in range(NBUF - 1):
      fire_gathers(jnp.int32(b), b)

    def outer(i, carry):
      for b in range(NBUF):
        g = i * NBUF + b
        wait_gathers(b)
        pltpu.async_copy(rows[b], out_slice(g), osem[b])

        gp = g + (NBUF - 1)
        bp = (b + NBUF - 1) % NBUF

        @pl.when(jnp.logical_and(gp < NCH, g >= 1))
        def _():
          wait_outwrite(g - 1, bp)

        @pl.when(gp < NCH)
        def _():
          fire_gathers(gp, bp)
      return carry

    lax.fori_loop(0, NCH // NBUF, outer, 0)

    for b in range(NBUF):
      wait_outwrite(NCH - NBUF + b, b)

  return gather


def _build_finish(B, S, D, N, DP):
  BB = 4
  BR = BB * S
  assert B % BB == 0

  def body(rows_ref, pos_ref, out_ref):
    x = rows_ref[:, :D].reshape(BB, S, D)
    out_ref[...] = x + pos_ref[...][None]

  return pl.pallas_call(
      body,
      grid=(B // BB,),
      in_specs=[
          pl.BlockSpec((BR, DP), lambda i: (i, 0)),
          pl.BlockSpec((S, D), lambda i: (0, 0)),
      ],
      out_specs=pl.BlockSpec((BB, S, D), lambda i: (i, 0, 0)),
      out_shape=jax.ShapeDtypeStruct((B, S, D), jnp.float32),
  )


def kernel(input_ids, token_table, position_table):
  B, S = input_ids.shape
  V, D = token_table.shape
  N = B * S
  info = plsc.get_sparse_core_info()
  gather = _build_gather(V, D, N, info.num_cores, info.num_subcores)
  finish = _build_finish(B, S, D, N, 2 * D)
  idx = jnp.reshape(input_ids, (N,)).astype(jnp.int32)
  staged = gather(token_table, idx)
  return finish(staged, position_table)


# speedup vs baseline: 2.0377x; 2.0377x over previous
import functools

import jax
import jax.numpy as jnp
from jax import lax
from jax.experimental import pallas as pl
from jax.experimental.pallas import tpu as pltpu
from jax.experimental.pallas import tpu_sc as plsc


def _build_repack(V, D):
  LB = 4096
  H = LB // 2
  NBLK = pl.cdiv(V, LB)
  V2P = NBLK * H

  def body(tin, tout):
    x = tin[...]
    lane = jax.lax.broadcasted_iota(jnp.int32, (D, D), 1)
    sub = jax.lax.broadcasted_iota(jnp.int32, (D, D), 0)
    eye = jnp.where(lane == sub, 1.0, 0.0).astype(jnp.float32)
    y = jax.lax.dot_general(
        x, eye, (((0,), (0,)), ((), ())),
        preferred_element_type=jnp.float32)
    tout[...] = jnp.concatenate([y[:H], y[H:]], axis=1)

  return pl.pallas_call(
      body,
      grid=(NBLK,),
      in_specs=[pl.BlockSpec((D, LB), lambda i: (0, i))],
      out_specs=pl.BlockSpec((H, 2 * D), lambda i: (i, 0)),
      out_shape=jax.ShapeDtypeStruct((V2P, 2 * D), jnp.float32),
  ), LB, V2P


def _permute_idx(ids, LB):
  H = LB // 2
  p = ids & (LB - 1)
  return (ids - p) + (p << 1) - jnp.where(p >= H, LB - 1, 0)


def _build_gather(D, N, S, NC, NS):
  NW = NC * NS
  NR = N // NW
  CH = 2 * S
  NCH = NR // CH
  NBUF = 4
  K = 80
  NG = CH // K
  REP = CH // S
  NL = 16
  DP = 2 * D
  HW = NW // 2
  assert N % NW == 0 and NR % CH == 0 and NCH % NBUF == 0 and CH % K == 0
  assert K % 8 == 0 and D % NL == 0

  mesh = plsc.VectorSubcoreMesh(core_axis_name="c", subcore_axis_name="s")

  scratch = (
      [pltpu.VMEM((CH, D), jnp.float32) for _ in range(NBUF)]
      + [pltpu.VMEM((CH,), jnp.int32) for _ in range(NBUF)]
      + [pltpu.VMEM((S, D), jnp.float32)]
      + [pltpu.SemaphoreType.DMA for _ in range(2 * NBUF)]
  )

  @functools.partial(
      pl.kernel,
      mesh=mesh,
      out_type=jax.ShapeDtypeStruct((N // 2, DP), jnp.float32),
      scratch_types=scratch,
      compiler_params=pltpu.CompilerParams(use_tc_tiling_on_sc=False),
  )
  def gather(table, idx_hbm, pos_hbm, out_hbm, *scr):
    rows = scr[0:NBUF]
    idxb = scr[NBUF:2 * NBUF]
    pos_v = scr[2 * NBUF]
    gsem = scr[2 * NBUF + 1: 3 * NBUF + 1]
    osem = scr[3 * NBUF + 1: 4 * NBUF + 1]

    wid = lax.axis_index("s") * NC + lax.axis_index("c")
    base = wid * NR
    high = wid >= HW
    obase = base - jnp.where(high, N // 2, 0)
    ocol = jnp.where(high, D, 0)

    pltpu.sync_copy(pos_hbm, pos_v)

    def fire_gathers(g, b):
      r0 = base + g * CH
      pltpu.sync_copy(idx_hbm.at[pl.ds(r0, CH)], idxb[b])
      for kk in range(NG):
        pltpu.async_copy(
            table.at[idxb[b].at[pl.ds(kk * K, K)]],
            rows[b].at[pl.ds(kk * K, K)],
            gsem[b],
        )

    def wait_gathers(b):
      for kk in range(NG):
        pltpu.make_async_copy(
            table.at[idxb[b].at[pl.ds(kk * K, K)]],
            rows[b].at[pl.ds(kk * K, K)],
            gsem[b],
        ).wait()

    def out_slice(g):
      return out_hbm.at[pl.ds(obase + g * CH, CH), pl.ds(ocol, D)]

    def wait_outwrite(g, b):
      pltpu.make_async_copy(rows[b], out_slice(g), osem[b]).wait()

    for b in range(NBUF - 1):
      fire_gathers(jnp.int32(b), b)

    def outer(i, carry):
      for b in range(NBUF):
        g = i * NBUF + b
        wait_gathers(b)

        def add_pos(j, c2, _rows=rows[b]):
          for c in range(D // NL):
            pv = pos_v[j, pl.ds(c * NL, NL)]
            for rep in range(REP):
              r = rep * S + j
              _rows[r, pl.ds(c * NL, NL)] = _rows[r, pl.ds(c * NL, NL)] + pv
          return c2
        lax.fori_loop(0, S, add_pos, 0)

        pltpu.async_copy(rows[b], out_slice(g), osem[b])

        gp = g + (NBUF - 1)
        bp = (b + NBUF - 1) % NBUF

        @pl.when(jnp.logical_and(gp < NCH, g >= 1))
        def _():
          wait_outwrite(g - 1, bp)

        @pl.when(gp < NCH)
        def _():
          fire_gathers(gp, bp)
      return carry

    lax.fori_loop(0, NCH // NBUF, outer, 0)

    for b in range(NBUF):
      wait_outwrite(NCH - NBUF + b, b)

  return gather


def _build_finish(B, S, D, DP):
  BB = 512
  SB = 8
  BH = B // 2
  GB = B // BB
  assert B % BB == 0 and S % SB == 0 and BH % BB == 0

  def body(tin, tout):
    x = tin[...]
    bid = pl.program_id(0)
    lane = jax.lax.broadcasted_iota(jnp.int32, (D, DP), 1)
    sub = jax.lax.broadcasted_iota(jnp.int32, (D, DP), 0)
    sel = jnp.where(bid < GB // 2, sub, sub + D)
    j2 = jnp.where(lane == sel, 1.0, 0.0).astype(jnp.float32)
    for s in range(SB):
      tout[s] = jax.lax.dot_general(
          j2, x[:, s, :], (((1,), (1,)), ((), ())),
          preferred_element_type=jnp.float32)

  return pl.pallas_call(
      body,
      grid=(GB, S // SB),
      in_specs=[pl.BlockSpec(
          (BB, SB, DP),
          lambda b, s: (jnp.where(b < GB // 2, b, b - GB // 2), s, 0))],
      out_specs=pl.BlockSpec((SB, D, BB), lambda b, s: (s, 0, b)),
      out_shape=jax.ShapeDtypeStruct((S, D, B), jnp.float32),
  )


def kernel(input_ids, token_table, position_table):
  B, S = input_ids.shape
  V, D = token_table.shape
  N = B * S
  info = plsc.get_sparse_core_info()
  repack, LB, V2P = _build_repack(V, D)
  gather = _build_gather(D, N, S, info.num_cores, info.num_subcores)
  finish = _build_finish(B, S, D, 2 * D)

  idx = _permute_idx(jnp.reshape(input_ids, (N,)).astype(jnp.int32), LB)
  table_lin = repack(token_table.T).reshape(2 * V2P, D)
  staged = gather(table_lin, idx, position_table)
  out_t = finish(staged.reshape(B // 2, S, 2 * D))
  return jnp.transpose(out_t, (2, 0, 1))
